# baseline (speedup 1.0000x reference)
import functools

import jax
import jax.numpy as jnp
from jax import lax
from jax.experimental import pallas as pl
from jax.experimental.pallas import tpu as pltpu


def _body(a_hbm, x_ref, w0_ref, w1_ref, o_ref,
          abuf, b0_ref, b1_ref, h_ref, sems, *, nsteps, block_m, nbuf):
    total = 2 * nsteps

    def issue(step, slot):
        row = lax.rem(step, nsteps) * block_m
        for q in range(2):
            pltpu.make_async_copy(
                a_hbm.at[q, pl.ds(row, block_m), :],
                abuf.at[slot, q],
                sems.at[slot],
            ).start()

    def wait(slot):
        for q in range(2):
            pltpu.make_async_copy(
                a_hbm.at[q, pl.ds(0, block_m), :],
                abuf.at[slot, q],
                sems.at[slot],
            ).wait()

    for s in range(nbuf - 1):
        issue(s, s)

    x = x_ref[...]
    b0_ref[0] = jnp.dot(x, w0_ref[0], preferred_element_type=jnp.float32)
    b0_ref[1] = jnp.dot(x, w0_ref[1], preferred_element_type=jnp.float32)

    def step_fn(s, carry):
        @pl.when(s + nbuf - 1 < total)
        def _():
            issue(s + nbuf - 1, lax.rem(s + nbuf - 1, nbuf))

        slot = lax.rem(s, nbuf)
        wait(slot)

        row = lax.rem(s, nsteps) * block_m

        @pl.when(s < nsteps)
        def _():
            acc = jnp.dot(abuf[slot, 0], b0_ref[0],
                          preferred_element_type=jnp.float32)
            acc = acc + jnp.dot(abuf[slot, 1], b0_ref[1],
                                preferred_element_type=jnp.float32)
            h_ref[pl.ds(row, block_m), :] = jnp.maximum(acc, 0.0).astype(
                jnp.bfloat16)

        @pl.when(s == nsteps)
        def _():
            h = h_ref[...]
            b1_ref[0] = jnp.dot(h, w1_ref[0].astype(jnp.bfloat16),
                                preferred_element_type=jnp.float32)
            b1_ref[1] = jnp.dot(h, w1_ref[1].astype(jnp.bfloat16),
                                preferred_element_type=jnp.float32)

        @pl.when(s >= nsteps)
        def _():
            acc = jnp.dot(abuf[slot, 0], b1_ref[0],
                          preferred_element_type=jnp.float32)
            acc = acc + jnp.dot(abuf[slot, 1], b1_ref[1],
                                preferred_element_type=jnp.float32)
            o_ref[pl.ds(row, block_m), :] = acc

        return carry

    lax.fori_loop(0, total, step_fn, 0)


def kernel(x, poly_ls, W0, W1):
    p, n, _ = poly_ls.shape
    d_in = x.shape[1]
    d_h = W0.shape[2]
    d_out = W1.shape[2]
    block_m = 80 if n % 80 == 0 else 8
    nbuf = 5
    nsteps = n // block_m
    return pl.pallas_call(
        functools.partial(_body, nsteps=nsteps, block_m=block_m, nbuf=nbuf),
        in_specs=[
            pl.BlockSpec(memory_space=pl.ANY),
            pl.BlockSpec(memory_space=pltpu.VMEM),
            pl.BlockSpec(memory_space=pltpu.VMEM),
            pl.BlockSpec(memory_space=pltpu.VMEM),
        ],
        out_specs=pl.BlockSpec(memory_space=pltpu.VMEM),
        out_shape=jax.ShapeDtypeStruct((n, d_out), jnp.float32),
        compiler_params=pltpu.CompilerParams(
            vmem_limit_bytes=64 * 1024 * 1024,
        ),
        scratch_shapes=[
            pltpu.VMEM((nbuf, p, block_m, n), jnp.float32),
            pltpu.VMEM((p, n, d_h), jnp.float32),
            pltpu.VMEM((p, n, d_out), jnp.float32),
            pltpu.VMEM((n, d_h), jnp.bfloat16),
            pltpu.SemaphoreType.DMA((nbuf,)),
        ],
    )(poly_ls, x, W0, W1)


# speedup vs baseline: 1.0037x; 1.0037x over previous
import functools

import jax
import jax.numpy as jnp
from jax import lax
from jax.experimental import pallas as pl
from jax.experimental.pallas import tpu as pltpu


def _body(a_hbm, x_ref, w0_ref, w1_ref, o_ref,
          abuf, b0_ref, b1_ref, h_ref, sems, *, nsteps, block_m, nbuf):
    total = 2 * nsteps

    def issue(step, slot):
        row = lax.rem(step, nsteps) * block_m
        pltpu.make_async_copy(
            a_hbm.at[:, pl.ds(row, block_m), :],
            abuf.at[slot],
            sems.at[slot],
        ).start()

    def wait(slot):
        pltpu.make_async_copy(
            a_hbm.at[:, pl.ds(0, block_m), :],
            abuf.at[slot],
            sems.at[slot],
        ).wait()

    for s in range(nbuf - 1):
        issue(s, s)

    x = x_ref[...]
    b0_ref[0] = jnp.dot(x, w0_ref[0], preferred_element_type=jnp.float32)
    b0_ref[1] = jnp.dot(x, w0_ref[1], preferred_element_type=jnp.float32)

    def step_fn(s, carry):
        @pl.when(s + nbuf - 1 < total)
        def _():
            issue(s + nbuf - 1, lax.rem(s + nbuf - 1, nbuf))

        slot = lax.rem(s, nbuf)
        wait(slot)

        row = lax.rem(s, nsteps) * block_m

        @pl.when(s < nsteps)
        def _():
            acc = jnp.dot(abuf[slot, 0], b0_ref[0],
                          preferred_element_type=jnp.float32)
            acc = acc + jnp.dot(abuf[slot, 1], b0_ref[1],
                                preferred_element_type=jnp.float32)
            h_ref[pl.ds(row, block_m), :] = jnp.maximum(acc, 0.0).astype(
                jnp.bfloat16)

        @pl.when(s == nsteps)
        def _():
            h = h_ref[...]
            b1_ref[0] = jnp.dot(h, w1_ref[0].astype(jnp.bfloat16),
                                preferred_element_type=jnp.float32)
            b1_ref[1] = jnp.dot(h, w1_ref[1].astype(jnp.bfloat16),
                                preferred_element_type=jnp.float32)

        @pl.when(s >= nsteps)
        def _():
            acc = jnp.dot(abuf[slot, 0], b1_ref[0],
                          preferred_element_type=jnp.float32)
            acc = acc + jnp.dot(abuf[slot, 1], b1_ref[1],
                                preferred_element_type=jnp.float32)
            o_ref[pl.ds(row, block_m), :] = acc

        return carry

    lax.fori_loop(0, total, step_fn, 0)


def kernel(x, poly_ls, W0, W1):
    p, n, _ = poly_ls.shape
    d_in = x.shape[1]
    d_h = W0.shape[2]
    d_out = W1.shape[2]
    block_m = 80 if n % 80 == 0 else 8
    nbuf = 5
    nsteps = n // block_m
    return pl.pallas_call(
        functools.partial(_body, nsteps=nsteps, block_m=block_m, nbuf=nbuf),
        in_specs=[
            pl.BlockSpec(memory_space=pl.ANY),
            pl.BlockSpec(memory_space=pltpu.VMEM),
            pl.BlockSpec(memory_space=pltpu.VMEM),
            pl.BlockSpec(memory_space=pltpu.VMEM),
        ],
        out_specs=pl.BlockSpec(memory_space=pltpu.VMEM),
        out_shape=jax.ShapeDtypeStruct((n, d_out), jnp.float32),
        compiler_params=pltpu.CompilerParams(
            vmem_limit_bytes=64 * 1024 * 1024,
        ),
        scratch_shapes=[
            pltpu.VMEM((nbuf, p, block_m, n), jnp.float32),
            pltpu.VMEM((p, n, d_h), jnp.float32),
            pltpu.VMEM((p, n, d_out), jnp.float32),
            pltpu.VMEM((n, d_h), jnp.bfloat16),
            pltpu.SemaphoreType.DMA((nbuf,)),
        ],
    )(poly_ls, x, W0, W1)


# speedup vs baseline: 1.0110x; 1.0073x over previous
import functools

import jax
import jax.numpy as jnp
from jax import lax
from jax.experimental import pallas as pl
from jax.experimental.pallas import tpu as pltpu


def _body(a_hbm, x_ref, w0_ref, w1_ref, o_ref,
          abuf, b0_ref, b1_ref, sems, *, nsteps, block_m, nbuf):
    total = 2 * nsteps

    def issue(step, slot):
        row = lax.rem(step, nsteps) * block_m
        pltpu.make_async_copy(
            a_hbm.at[:, pl.ds(row, block_m), :],
            abuf.at[slot],
            sems.at[slot],
        ).start()

    def wait(slot):
        pltpu.make_async_copy(
            a_hbm.at[:, pl.ds(0, block_m), :],
            abuf.at[slot],
            sems.at[slot],
        ).wait()

    for s in range(nbuf - 1):
        issue(s, s)

    x = x_ref[...]
    b0_ref[0] = jnp.dot(x, w0_ref[0], preferred_element_type=jnp.float32)
    b0_ref[1] = jnp.dot(x, w0_ref[1], preferred_element_type=jnp.float32)
    w1_bf = w1_ref[...].astype(jnp.bfloat16)

    def step_fn(s, carry):
        @pl.when(s + nbuf - 1 < total)
        def _():
            issue(s + nbuf - 1, lax.rem(s + nbuf - 1, nbuf))

        slot = lax.rem(s, nbuf)
        wait(slot)
        row = lax.rem(s, nsteps) * block_m

        @pl.when(s < nsteps)
        def _():
            acc = jnp.dot(abuf[slot, 0], b0_ref[0],
                          preferred_element_type=jnp.float32)
            acc = acc + jnp.dot(abuf[slot, 1], b0_ref[1],
                                preferred_element_type=jnp.float32)
            hblk = jnp.maximum(acc, 0.0).astype(jnp.bfloat16)
            b1_ref[0, pl.ds(row, block_m), :] = jnp.dot(
                hblk, w1_bf[0], preferred_element_type=jnp.float32)
            b1_ref[1, pl.ds(row, block_m), :] = jnp.dot(
                hblk, w1_bf[1], preferred_element_type=jnp.float32)

        @pl.when(s >= nsteps)
        def _():
            acc = jnp.dot(abuf[slot, 0], b1_ref[0],
                          preferred_element_type=jnp.float32)
            acc = acc + jnp.dot(abuf[slot, 1], b1_ref[1],
                                preferred_element_type=jnp.float32)
            o_ref[pl.ds(row, block_m), :] = acc

        return carry

    lax.fori_loop(0, total, step_fn, 0)


def kernel(x, poly_ls, W0, W1):
    p, n, _ = poly_ls.shape
    d_in = x.shape[1]
    d_h = W0.shape[2]
    d_out = W1.shape[2]
    block_m = 80 if n % 80 == 0 else 8
    nbuf = 4
    nsteps = n // block_m
    return pl.pallas_call(
        functools.partial(_body, nsteps=nsteps, block_m=block_m, nbuf=nbuf),
        in_specs=[
            pl.BlockSpec(memory_space=pl.ANY),
            pl.BlockSpec(memory_space=pltpu.VMEM),
            pl.BlockSpec(memory_space=pltpu.VMEM),
            pl.BlockSpec(memory_space=pltpu.VMEM),
        ],
        out_specs=pl.BlockSpec(memory_space=pltpu.VMEM),
        out_shape=jax.ShapeDtypeStruct((n, d_out), jnp.float32),
        scratch_shapes=[
            pltpu.VMEM((nbuf, p, block_m, n), jnp.float32),
            pltpu.VMEM((p, n, d_h), jnp.float32),
            pltpu.VMEM((p, n, d_out), jnp.float32),
            pltpu.SemaphoreType.DMA((nbuf,)),
        ],
    )(poly_ls, x, W0, W1)
